# baseline (speedup 1.0000x reference)
import math

import jax
import jax.numpy as jnp
from jax.experimental import pallas as pl
from jax.experimental.pallas import tpu as pltpu

TEMPERATURE = 0.1
BLOCK_Q = 512


def _attn_body(z_q_ref, z_ref, wq_ref, bq_ref, wk_ref, bk_ref, wv_ref, bv_ref,
               out_ref, k_scr, v_scr):
    i = pl.program_id(0)

    @pl.when(i == 0)
    def _project_kv():
        zf = z_ref[...].astype(jnp.bfloat16)
        k = jax.lax.dot_general(
            zf, wk_ref[...], (((1,), (1,)), ((), ())),
            preferred_element_type=jnp.float32) + bk_ref[...]
        k_scr[...] = k.astype(jnp.bfloat16)
        v = jax.lax.dot_general(
            zf, wv_ref[...], (((1,), (1,)), ((), ())),
            preferred_element_type=jnp.float32) + bv_ref[...]
        v_scr[...] = v.astype(jnp.bfloat16)

    z_q = z_q_ref[...]
    q = jax.lax.dot_general(
        z_q.astype(jnp.bfloat16), wq_ref[...], (((1,), (1,)), ((), ())),
        preferred_element_type=jnp.float32) + bq_ref[...]
    s = jax.lax.dot_general(
        q.astype(jnp.bfloat16), k_scr[...], (((1,), (1,)), ((), ())),
        preferred_element_type=jnp.float32)
    m = jnp.max(s, axis=1, keepdims=True)
    p = jnp.exp2(s - m)
    denom = jnp.sum(p, axis=1, keepdims=True)
    o = jax.lax.dot_general(
        p.astype(jnp.bfloat16), v_scr[...], (((1,), (0,)), ((), ())),
        preferred_element_type=jnp.float32)
    out_ref[...] = z_q + o * (0.5 / denom)


def kernel(z, labels, Wq, bq, Wk, bk, Wv, bv, memory, memory_weights,
           memory_labels, running_mean, running_cov):
    B, d = z.shape
    c = math.log2(math.e) / (math.sqrt(d) * TEMPERATURE)
    wq16 = (Wq * c).astype(jnp.bfloat16)
    bq_s = (bq * c).reshape(1, d)
    wk16 = Wk.astype(jnp.bfloat16)
    wv16 = Wv.astype(jnp.bfloat16)
    bk2 = bk.reshape(1, d)
    bv2 = bv.reshape(1, d)
    nq = B // BLOCK_Q
    full = lambda i: (0, 0)
    out = pl.pallas_call(
        _attn_body,
        grid=(nq,),
        in_specs=[
            pl.BlockSpec((BLOCK_Q, d), lambda i: (i, 0)),
            pl.BlockSpec((B, d), full),
            pl.BlockSpec((d, d), full),
            pl.BlockSpec((1, d), full),
            pl.BlockSpec((d, d), full),
            pl.BlockSpec((1, d), full),
            pl.BlockSpec((d, d), full),
            pl.BlockSpec((1, d), full),
        ],
        out_specs=pl.BlockSpec((BLOCK_Q, d), lambda i: (i, 0)),
        out_shape=jax.ShapeDtypeStruct((B, d), jnp.float32),
        scratch_shapes=[
            pltpu.VMEM((B, d), jnp.bfloat16),
            pltpu.VMEM((B, d), jnp.bfloat16),
        ],
    )(z, z, wq16, bq_s, wk16, bk2, wv16, bv2)
    return out


# speedup vs baseline: 1.5950x; 1.5950x over previous
import math

import jax
import jax.numpy as jnp
from jax.experimental import pallas as pl
from jax.experimental.pallas import tpu as pltpu

TEMPERATURE = 0.1
BLOCK_Q = 4096


def _attn_body(z_q_ref, z_ref, wq_ref, bq_ref, wk_ref, bk_ref, wv_ref, bv_ref,
               out_ref, k_scr, v_scr):
    i = pl.program_id(0)

    @pl.when(i == 0)
    def _project_kv():
        zf = z_ref[...].astype(jnp.bfloat16)
        k = jax.lax.dot_general(
            zf, wk_ref[...], (((1,), (1,)), ((), ())),
            preferred_element_type=jnp.float32) + bk_ref[...]
        k_scr[...] = k.astype(jnp.bfloat16)
        v = jax.lax.dot_general(
            zf, wv_ref[...], (((1,), (1,)), ((), ())),
            preferred_element_type=jnp.float32) + bv_ref[...]
        v_scr[...] = v.astype(jnp.bfloat16)

    h = BLOCK_Q // 8
    for hb in range(8):
        sl = pl.ds(hb * h, h)
        z_q = z_q_ref[sl, :]
        q = jax.lax.dot_general(
            z_q.astype(jnp.bfloat16), wq_ref[...], (((1,), (1,)), ((), ())),
            preferred_element_type=jnp.float32) + bq_ref[...]
        s = jax.lax.dot_general(
            q.astype(jnp.bfloat16), k_scr[...], (((1,), (1,)), ((), ())),
            preferred_element_type=jnp.float32)
        m = jnp.max(s, axis=1, keepdims=True)
        p = jnp.exp2(s - m)
        denom = jnp.sum(p, axis=1, keepdims=True)
        o = jax.lax.dot_general(
            p.astype(jnp.bfloat16), v_scr[...], (((1,), (0,)), ((), ())),
            preferred_element_type=jnp.float32)
        out_ref[sl, :] = z_q + o * (0.5 / denom)


def kernel(z, labels, Wq, bq, Wk, bk, Wv, bv, memory, memory_weights,
           memory_labels, running_mean, running_cov):
    B, d = z.shape
    c = math.log2(math.e) / (math.sqrt(d) * TEMPERATURE)
    wq16 = (Wq * c).astype(jnp.bfloat16)
    bq_s = (bq * c).reshape(1, d)
    wk16 = Wk.astype(jnp.bfloat16)
    wv16 = Wv.astype(jnp.bfloat16)
    bk2 = bk.reshape(1, d)
    bv2 = bv.reshape(1, d)
    nq = B // BLOCK_Q
    full = lambda i: (0, 0)
    out = pl.pallas_call(
        _attn_body,
        grid=(nq,),
        in_specs=[
            pl.BlockSpec((BLOCK_Q, d), lambda i: (i, 0)),
            pl.BlockSpec((B, d), full),
            pl.BlockSpec((d, d), full),
            pl.BlockSpec((1, d), full),
            pl.BlockSpec((d, d), full),
            pl.BlockSpec((1, d), full),
            pl.BlockSpec((d, d), full),
            pl.BlockSpec((1, d), full),
        ],
        out_specs=pl.BlockSpec((BLOCK_Q, d), lambda i: (i, 0)),
        out_shape=jax.ShapeDtypeStruct((B, d), jnp.float32),
        scratch_shapes=[
            pltpu.VMEM((B, d), jnp.bfloat16),
            pltpu.VMEM((B, d), jnp.bfloat16),
        ],
    )(z, z, wq16, bq_s, wk16, bk2, wv16, bv2)
    return out
